# baseline (speedup 1.0000x reference)
import functools

import jax
import jax.numpy as jnp
from jax import lax
from jax.experimental import pallas as pl
from jax.experimental.pallas import tpu as pltpu
from jax.experimental.pallas import tpu_sc as plsc

BATCH = 16384
ARITY = 2
EMBED_DIM = 64
HIDDEN = 256
NUM_CONST = 1000000

NC, NS = 2, 16
NW = NC * NS
BW = BATCH // NW
QCHUNK = 128
NQ = BW // QCHUNK
FDIM = ARITY * EMBED_DIM


def _sc_gather(tbl_flat, idxT):
    mesh = plsc.VectorSubcoreMesh(core_axis_name="c", subcore_axis_name="s")

    @functools.partial(
        pl.kernel,
        out_type=jax.ShapeDtypeStruct((FDIM, BATCH), jnp.float32),
        mesh=mesh,
        scratch_types=[
            pltpu.VMEM((ARITY, BW), jnp.int32),
            pltpu.VMEM((EMBED_DIM, QCHUNK), jnp.int32),
            pltpu.VMEM((FDIM, BW), jnp.float32),
            pltpu.SemaphoreType.DMA,
        ],
        compiler_params=pltpu.CompilerParams(needs_layout_passes=False),
    )
    def gather_kernel(tbl_hbm, idx_hbm, out_hbm, idx_v, fidx_v, xT_v, sem):
        wid = lax.axis_index("s") * NC + lax.axis_index("c")
        b0 = wid * BW
        for a in range(ARITY):
            pltpu.sync_copy(idx_hbm.at[a, pl.ds(b0, BW)], idx_v.at[a])

        def q_body(q, carry):
            for a in range(ARITY):
                iv = [idx_v[a, pl.ds(q * QCHUNK + 16 * k, 16)]
                      for k in range(QCHUNK // 16)]
                for c in range(EMBED_DIM):
                    for k in range(QCHUNK // 16):
                        fidx_v[c, pl.ds(16 * k, 16)] = iv[k] + (c * NUM_CONST)
                copies = []
                for c in range(EMBED_DIM):
                    copies.append(
                        pltpu.async_copy(
                            tbl_hbm.at[fidx_v.at[c]],
                            xT_v.at[a * EMBED_DIM + c, pl.ds(q * QCHUNK, QCHUNK)],
                            sem,
                        )
                    )
                for cp in copies:
                    cp.wait()
            return carry

        lax.fori_loop(0, NQ, q_body, 0)
        pltpu.sync_copy(xT_v, out_hbm.at[:, pl.ds(b0, BW)])

    return gather_kernel(tbl_flat, idxT)


def _mlp_body(x_ref, w1t_ref, b1_ref, w2_ref, b2_ref, o_ref):
    xb = x_ref[...].astype(jnp.bfloat16)
    w1b = w1t_ref[...].astype(jnp.bfloat16)
    h = jnp.dot(w1b, xb, preferred_element_type=jnp.float32)
    h = jnp.maximum(h + b1_ref[...][:, None], 0.0)
    logit = jnp.sum(h * w2_ref[...][:, None], axis=0) + b2_ref[0]
    o_ref[...] = jax.nn.sigmoid(logit)


def _tc_mlp(xT, W1t, b1, W2c, b2):
    TB = 1024
    grid = (BATCH // TB,)
    return pl.pallas_call(
        _mlp_body,
        grid=grid,
        in_specs=[
            pl.BlockSpec((FDIM, TB), lambda i: (0, i)),
            pl.BlockSpec((HIDDEN, FDIM), lambda i: (0, 0)),
            pl.BlockSpec((HIDDEN,), lambda i: (0,)),
            pl.BlockSpec((HIDDEN,), lambda i: (0,)),
            pl.BlockSpec(memory_space=pltpu.SMEM),
        ],
        out_specs=pl.BlockSpec((TB,), lambda i: (i,)),
        out_shape=jax.ShapeDtypeStruct((BATCH,), jnp.float32),
    )(xT, W1t, b1, W2c, b2)


def kernel(indices, table, W1, b1, W2, b2):
    idxT = indices.astype(jnp.int32).T
    tbl_flat = table.T.reshape(NUM_CONST * EMBED_DIM)
    xT = _sc_gather(tbl_flat, idxT)
    return _tc_mlp(xT, W1.T, b1, W2.reshape(HIDDEN), b2)


# speedup vs baseline: 18.4091x; 18.4091x over previous
import functools

import jax
import jax.numpy as jnp
from jax import lax
from jax.experimental import pallas as pl
from jax.experimental.pallas import tpu as pltpu
from jax.experimental.pallas import tpu_sc as plsc

BATCH = 16384
ARITY = 2
EMBED_DIM = 64
HIDDEN = 256
NUM_CONST = 1000000

ROWS = BATCH * ARITY
NC, NS = 2, 16
NW = NC * NS
ROWS_PER_W = ROWS // NW
GROUP = 8
FIRE = 32
NBATCH = ROWS_PER_W // FIRE


def _sc_gather(tbl3, idx2):
    mesh = plsc.VectorSubcoreMesh(core_axis_name="c", subcore_axis_name="s")

    @functools.partial(
        pl.kernel,
        out_type=jax.ShapeDtypeStruct((BATCH, ARITY * EMBED_DIM), jnp.float32),
        mesh=mesh,
        scratch_types=[
            pltpu.VMEM((ROWS_PER_W,), jnp.int32),
            pltpu.VMEM((ROWS_PER_W // 2, ARITY * EMBED_DIM), jnp.float32),
            pltpu.SemaphoreType.DMA,
        ],
        compiler_params=pltpu.CompilerParams(needs_layout_passes=False),
    )
    def gather_kernel(tbl_hbm, idx_hbm, out_hbm, idx_v, rows_v, sem):
        wid = lax.axis_index("s") * NC + lax.axis_index("c")
        pltpu.sync_copy(idx_hbm.at[wid], idx_v)
        out_base = wid * (ROWS_PER_W // 2)

        def batch_body(b, carry):
            copies = []
            for k2 in range(FIRE // 16):
                iv = idx_v[pl.ds(b * FIRE + k2 * 16, 16)]
                for k1 in range(16):
                    k = k2 * 16 + k1
                    v = iv[k1]
                    copies.append(
                        pltpu.async_copy(
                            tbl_hbm.at[jnp.right_shift(v, 3),
                                       jnp.bitwise_and(v, 7)],
                            rows_v.at[b * (FIRE // 2) + k // 2,
                                      pl.ds((k % 2) * EMBED_DIM, EMBED_DIM)],
                            sem,
                        )
                    )
            for c in copies:
                c.wait()
            return carry

        lax.fori_loop(0, NBATCH, batch_body, 0)
        pltpu.sync_copy(
            rows_v, out_hbm.at[pl.ds(out_base, ROWS_PER_W // 2)]
        )

    return gather_kernel(tbl3, idx2)


def _mlp_body(x_ref, w1_ref, b1_ref, w2t_ref, b2_ref, o_ref):
    xb = x_ref[...].astype(jnp.bfloat16)
    w1b = w1_ref[...].astype(jnp.bfloat16)
    h = jnp.dot(xb, w1b, preferred_element_type=jnp.float32)
    h = jnp.maximum(h + b1_ref[...][None, :], 0.0)
    logit = jnp.sum(h * w2t_ref[...], axis=1) + b2_ref[0]
    o_ref[...] = jax.nn.sigmoid(logit)


def _tc_mlp(x, W1, b1, W2t, b2):
    TB = 1024
    grid = (BATCH // TB,)
    return pl.pallas_call(
        _mlp_body,
        grid=grid,
        in_specs=[
            pl.BlockSpec((TB, ARITY * EMBED_DIM), lambda i: (i, 0)),
            pl.BlockSpec((ARITY * EMBED_DIM, HIDDEN), lambda i: (0, 0)),
            pl.BlockSpec((HIDDEN,), lambda i: (0,)),
            pl.BlockSpec((1, HIDDEN), lambda i: (0, 0)),
            pl.BlockSpec(memory_space=pltpu.SMEM),
        ],
        out_specs=pl.BlockSpec((TB,), lambda i: (i,)),
        out_shape=jax.ShapeDtypeStruct((BATCH,), jnp.float32),
    )(x, W1, b1, W2t, b2)


def kernel(indices, table, W1, b1, W2, b2):
    idx2 = indices.astype(jnp.int32).reshape(NW, ROWS_PER_W)
    tbl3 = table.reshape(NUM_CONST // GROUP, GROUP, EMBED_DIM)
    x = _sc_gather(tbl3, idx2)
    return _tc_mlp(x, W1, b1, W2.reshape(1, HIDDEN), b2)
